# baseline (speedup 1.0000x reference)
import functools

import jax
import jax.numpy as jnp
from jax import lax
from jax.experimental import pallas as pl
from jax.experimental.pallas import tpu as pltpu
from jax.experimental.pallas import tpu_sc as plsc

B, N, L = 1024, 26, 20
V, D, H = 100000, 768, 512
HH = H // 2
S = B * N

NC, NS, LANES = 2, 16, 16
NW = NC * NS
SPW = S // NW
CHSC = 4


def _proj_matmul(wte, Wfc):
  BM = 2000

  def body(wte_ref, wfc_ref, out_ref):
    out_ref[...] = jnp.dot(wte_ref[...], wfc_ref[...],
                           preferred_element_type=jnp.float32,
                           precision=lax.Precision.HIGHEST)

  return pl.pallas_call(
      body,
      grid=(V // BM,),
      in_specs=[
          pl.BlockSpec((BM, D), lambda i: (i, 0)),
          pl.BlockSpec((D, HH), lambda i: (0, 0)),
      ],
      out_specs=pl.BlockSpec((BM, HH), lambda i: (i, 0)),
      out_shape=jax.ShapeDtypeStruct((V, HH), jnp.float32),
  )(wte, Wfc)


def _lane_bcast(vec, lane):
  idx = jnp.full((LANES, 1), lane, jnp.int32)
  dnums = lax.GatherDimensionNumbers(
      offset_dims=(), collapsed_slice_dims=(0,), start_index_map=(0,))
  return lax.gather(vec, idx, dnums, (1,),
                    mode=lax.GatherScatterMode.PROMISE_IN_BOUNDS)


def _sc_gather_pool(proj, tokf, maskf):
  mesh = plsc.VectorSubcoreMesh(core_axis_name="c", subcore_axis_name="s")
  NCH = SPW // CHSC

  @functools.partial(
      pl.kernel,
      out_type=jax.ShapeDtypeStruct((S, 2, 128), jnp.float32),
      mesh=mesh,
      scratch_types=[
          pltpu.VMEM((SPW * L,), jnp.int32),
          pltpu.VMEM((SPW * L + 8,), jnp.float32),
          pltpu.VMEM((CHSC * L, 2, 128), jnp.float32),
          pltpu.VMEM((CHSC * L, 2, 128), jnp.float32),
          pltpu.VMEM((2 * CHSC, 2, 128), jnp.float32),
          pltpu.SemaphoreType.DMA,
          pltpu.SemaphoreType.DMA,
      ],
  )
  def sc_kernel(proj_hbm, tok_hbm, mask_hbm, out_hbm,
                tok_v, mask_v, rows_a, rows_b, ostage, sem_a, sem_b):
    wid = lax.axis_index("s") * NC + lax.axis_index("c")
    base = wid * SPW
    pltpu.sync_copy(tok_hbm.at[pl.ds(base * L, SPW * L)], tok_v)
    pltpu.sync_copy(mask_hbm.at[pl.ds(base * L, SPW * L)],
                    mask_v.at[pl.ds(0, SPW * L)])

    HLF = CHSC * L // 2

    def issue(c, buf, sem):
      for k in range(2):
        idx = tok_v.at[pl.ds(c * (CHSC * L) + k * HLF, HLF)]
        pltpu.async_copy(proj_hbm.at[idx], buf.at[pl.ds(k * HLF, HLF)], sem)

    def drain(buf, sem):
      for k in range(2):
        idx = tok_v.at[pl.ds(0, HLF)]
        pltpu.make_async_copy(proj_hbm.at[idx],
                              buf.at[pl.ds(k * HLF, HLF)], sem).wait()

    def compute(c, buf, ost_off):
      def pair_seg_body(pr, carry):
        q = (c * CHSC + 2 * pr) * L
        v0 = mask_v[pl.ds(q, LANES)]
        v1 = mask_v[pl.ds(q + LANES, LANES)]
        v2 = mask_v[pl.ds(q + 2 * LANES, LANES)]
        for half in range(2):
          acc = [jnp.zeros((LANES,), jnp.float32) for _ in range(HH // LANES)]
          for l in range(L):
            if half == 0:
              w = _lane_bcast(v0, l) if l < LANES                   else _lane_bcast(v1, l - LANES)
            else:
              w = _lane_bcast(v1, l + 4) if l < 12                   else _lane_bcast(v2, l - 12)
            row = (2 * pr + half) * L + l
            for j in range(HH // LANES):
              acc[j] = acc[j] + w * buf[row, j // 8,
                                        pl.ds((j % 8) * LANES, LANES)]
          for j in range(HH // LANES):
            ostage[ost_off + 2 * pr + half, j // 8,
                   pl.ds((j % 8) * LANES, LANES)] = acc[j]
        return carry

      lax.fori_loop(0, CHSC // 2, pair_seg_body, 0)

    issue(0, rows_a, sem_a)
    issue(1, rows_b, sem_b)

    def pair_body(p, carry):
      ca = 2 * p
      drain(rows_a, sem_a)
      compute(ca, rows_a, 0)
      issue(ca + 2, rows_a, sem_a)
      drain(rows_b, sem_b)
      compute(ca + 1, rows_b, CHSC)
      issue(ca + 3, rows_b, sem_b)
      pltpu.sync_copy(ostage, out_hbm.at[pl.ds(base + ca * CHSC, 2 * CHSC)])
      return carry

    lax.fori_loop(0, NCH // 2 - 1, pair_body, 0)
    drain(rows_a, sem_a)
    compute(NCH - 2, rows_a, 0)
    drain(rows_b, sem_b)
    compute(NCH - 1, rows_b, CHSC)
    pltpu.sync_copy(ostage,
                    out_hbm.at[pl.ds(base + (NCH - 2) * CHSC, 2 * CHSC)])

  return sc_kernel(proj.reshape(V, 2, 128), tokf, maskf).reshape(S, HH)


def _final_stage(ssum, mask2, coord2, bfc, W1, b1, W2, b2, Wc, bc):
  R = 512

  def body(ssum_ref, mask_ref, coord_ref, bfc_ref, w1_ref, b1_ref,
           w2_ref, b2_ref, wc_ref, bc_ref, out_ref):
    d = jnp.sum(mask_ref[...], axis=1, keepdims=True) + 1e-9
    cls = ssum_ref[...] / d + bfc_ref[...]
    c = coord_ref[...]
    w1 = w1_ref[...]
    h = (c[:, 0:1] * w1[0:1, :] + c[:, 1:2] * w1[1:2, :]
         + c[:, 2:3] * w1[2:3, :] + b1_ref[...])
    h = jnp.maximum(h, 0.0)
    ce = jnp.dot(h, w2_ref[...], preferred_element_type=jnp.float32,
                 precision=lax.Precision.HIGHEST) + b2_ref[...]
    wc = wc_ref[...]
    out_ref[...] = (
        jnp.dot(jnp.maximum(cls, 0.0), wc[:HH],
                preferred_element_type=jnp.float32,
                precision=lax.Precision.HIGHEST)
        + jnp.dot(jnp.maximum(ce, 0.0), wc[HH:],
                  preferred_element_type=jnp.float32,
                  precision=lax.Precision.HIGHEST)
        + bc_ref[...])

  return pl.pallas_call(
      body,
      grid=(S // R,),
      in_specs=[
          pl.BlockSpec((R, HH), lambda i: (i, 0)),
          pl.BlockSpec((R, L), lambda i: (i, 0)),
          pl.BlockSpec((R, 3), lambda i: (i, 0)),
          pl.BlockSpec((1, HH), lambda i: (0, 0)),
          pl.BlockSpec((3, HH), lambda i: (0, 0)),
          pl.BlockSpec((1, HH), lambda i: (0, 0)),
          pl.BlockSpec((HH, HH), lambda i: (0, 0)),
          pl.BlockSpec((1, HH), lambda i: (0, 0)),
          pl.BlockSpec((H, D), lambda i: (0, 0)),
          pl.BlockSpec((1, D), lambda i: (0, 0)),
      ],
      out_specs=pl.BlockSpec((R, D), lambda i: (i, 0)),
      out_shape=jax.ShapeDtypeStruct((S, D), jnp.float32),
  )(ssum, mask2, coord2, bfc.reshape(1, HH), W1, b1.reshape(1, HH),
    W2, b2.reshape(1, HH), Wc, bc.reshape(1, D))


def kernel(input_obs_node_gpt2_token, input_obs_node_gpt2_token_mask,
           input_obs_char_obj_rel_gpt2_token, wte, Wfc, bfc,
           W1, b1, W2, b2, Wc, bc):
  tok2 = input_obs_node_gpt2_token.reshape(S, L).astype(jnp.int32)
  mask2 = input_obs_node_gpt2_token_mask.reshape(S, L)
  coord2 = input_obs_char_obj_rel_gpt2_token.reshape(S, 3)

  proj = _proj_matmul(wte, Wfc)
  ssum = _sc_gather_pool(proj, tok2.reshape(-1), mask2.reshape(-1))
  out = _final_stage(ssum, mask2, coord2, bfc, W1, b1, W2, b2, Wc, bc)
  return out.reshape(B, N, D)


# speedup vs baseline: 1.1894x; 1.1894x over previous
import functools

import jax
import jax.numpy as jnp
from jax import lax
from jax.experimental import pallas as pl
from jax.experimental.pallas import tpu as pltpu
from jax.experimental.pallas import tpu_sc as plsc

B, N, L = 1024, 26, 20
V, D, H = 100000, 768, 512
HH = H // 2
S = B * N

NC, NS, LANES = 2, 16, 16
NW = NC * NS
SPW = S // NW
CHSC = 4


def _proj_matmul(wte, Wfc):
  BM = 2000

  def body(wte_ref, wfc_ref, out_ref):
    out_ref[...] = jnp.dot(
        wte_ref[...], wfc_ref[...], preferred_element_type=jnp.float32,
        precision=lax.Precision.HIGHEST).reshape(BM, 2, 128)

  return pl.pallas_call(
      body,
      grid=(V // BM,),
      in_specs=[
          pl.BlockSpec((BM, D), lambda i: (i, 0)),
          pl.BlockSpec((D, HH), lambda i: (0, 0)),
      ],
      out_specs=pl.BlockSpec((BM, 2, 128), lambda i: (i, 0, 0)),
      out_shape=jax.ShapeDtypeStruct((V, 2, 128), jnp.float32),
  )(wte, Wfc)


def _lane_bcast(vec, lane):
  idx = jnp.full((LANES, 1), lane, jnp.int32)
  dnums = lax.GatherDimensionNumbers(
      offset_dims=(), collapsed_slice_dims=(0,), start_index_map=(0,))
  return lax.gather(vec, idx, dnums, (1,),
                    mode=lax.GatherScatterMode.PROMISE_IN_BOUNDS)


def _sc_gather_pool(proj3, tok24f, maskf):
  mesh = plsc.VectorSubcoreMesh(core_axis_name="c", subcore_axis_name="s")
  NCH = SPW // CHSC

  @functools.partial(
      pl.kernel,
      out_type=jax.ShapeDtypeStruct((S, 2, 128), jnp.float32),
      mesh=mesh,
      scratch_types=[
          pltpu.VMEM((SPW * 24,), jnp.int32),
          pltpu.VMEM((SPW * 24 + 8,), jnp.float32),
          pltpu.VMEM((CHSC, L, 2, 128), jnp.float32),
          pltpu.VMEM((CHSC, L, 2, 128), jnp.float32),
          pltpu.VMEM((CHSC, 2, 128), jnp.float32),
          pltpu.SemaphoreType.DMA,
          pltpu.SemaphoreType.DMA,
      ],
  )
  def sc_kernel(proj_hbm, tok_hbm, mask_hbm, out_hbm,
                tok_v, mask_v, rows_a, rows_b, ostage, sem_a, sem_b):
    wid = lax.axis_index("s") * NC + lax.axis_index("c")
    base = wid * SPW
    pltpu.sync_copy(tok_hbm.at[pl.ds(base * 24, SPW * 24)], tok_v)
    pltpu.sync_copy(mask_hbm.at[pl.ds(base * 24, SPW * 24)],
                    mask_v.at[pl.ds(0, SPW * 24)])

    def issue(c, buf, sem):
      def body(k, carry):
        idx = tok_v.at[pl.ds((c * CHSC + k) * 24, L)]
        pltpu.async_copy(proj_hbm.at[idx], buf.at[k], sem)
        return carry
      lax.fori_loop(0, CHSC, body, 0)

    def drain(buf, sem):
      def body(k, carry):
        idx = tok_v.at[pl.ds(0, L)]
        pltpu.make_async_copy(proj_hbm.at[idx], buf.at[k], sem).wait()
        return carry
      lax.fori_loop(0, CHSC, body, 0)

    def compute(c, buf):
      def seg_body(si, carry):
        moff = (c * CHSC + si) * 24
        m0 = mask_v[pl.ds(moff, LANES)]
        m1 = mask_v[pl.ds(moff + LANES, LANES)]
        acc = [jnp.zeros((LANES,), jnp.float32) for _ in range(HH // LANES)]
        for l in range(L):
          if l < LANES:
            w = _lane_bcast(m0, l)
          else:
            w = _lane_bcast(m1, l - LANES)
          for j in range(HH // LANES):
            acc[j] = acc[j] + w * buf[si, l, j // 8,
                                      pl.ds((j % 8) * LANES, LANES)]
        for j in range(HH // LANES):
          ostage[si, j // 8, pl.ds((j % 8) * LANES, LANES)] = acc[j]
        return carry

      lax.fori_loop(0, CHSC, seg_body, 0)
      pltpu.sync_copy(ostage, out_hbm.at[pl.ds(base + c * CHSC, CHSC)])

    issue(0, rows_a, sem_a)
    issue(1, rows_b, sem_b)

    def pair_body(p, carry):
      ca = 2 * p
      drain(rows_a, sem_a)
      compute(ca, rows_a)
      issue(ca + 2, rows_a, sem_a)
      drain(rows_b, sem_b)
      compute(ca + 1, rows_b)
      issue(ca + 3, rows_b, sem_b)
      return carry

    lax.fori_loop(0, NCH // 2 - 1, pair_body, 0)
    drain(rows_a, sem_a)
    compute(NCH - 2, rows_a)
    drain(rows_b, sem_b)
    compute(NCH - 1, rows_b)

  return sc_kernel(proj3, tok24f, maskf)


def _final_stage(ssum3, mask2, coord2, bfc, W1, b1, W2, b2, Wc, bc):
  R = 512

  def body(ssum_ref, mask_ref, coord_ref, bfc_ref, w1_ref, b1_ref,
           w2_ref, b2_ref, wc_ref, bc_ref, out_ref):
    d = jnp.sum(mask_ref[...], axis=1, keepdims=True) + 1e-9
    cls = ssum_ref[...].reshape(R, HH) / d + bfc_ref[...]
    c = coord_ref[...]
    w1 = w1_ref[...]
    h = (c[:, 0:1] * w1[0:1, :] + c[:, 1:2] * w1[1:2, :]
         + c[:, 2:3] * w1[2:3, :] + b1_ref[...])
    h = jnp.maximum(h, 0.0)
    ce = jnp.dot(h, w2_ref[...], preferred_element_type=jnp.float32,
                 precision=lax.Precision.HIGHEST) + b2_ref[...]
    wc = wc_ref[...]
    out_ref[...] = (
        jnp.dot(jnp.maximum(cls, 0.0), wc[:HH],
                preferred_element_type=jnp.float32,
                precision=lax.Precision.HIGHEST)
        + jnp.dot(jnp.maximum(ce, 0.0), wc[HH:],
                  preferred_element_type=jnp.float32,
                  precision=lax.Precision.HIGHEST)
        + bc_ref[...])

  return pl.pallas_call(
      body,
      grid=(S // R,),
      in_specs=[
          pl.BlockSpec((R, 2, 128), lambda i: (i, 0, 0)),
          pl.BlockSpec((R, L), lambda i: (i, 0)),
          pl.BlockSpec((R, 3), lambda i: (i, 0)),
          pl.BlockSpec((1, HH), lambda i: (0, 0)),
          pl.BlockSpec((3, HH), lambda i: (0, 0)),
          pl.BlockSpec((1, HH), lambda i: (0, 0)),
          pl.BlockSpec((HH, HH), lambda i: (0, 0)),
          pl.BlockSpec((1, HH), lambda i: (0, 0)),
          pl.BlockSpec((H, D), lambda i: (0, 0)),
          pl.BlockSpec((1, D), lambda i: (0, 0)),
      ],
      out_specs=pl.BlockSpec((R, D), lambda i: (i, 0)),
      out_shape=jax.ShapeDtypeStruct((S, D), jnp.float32),
  )(ssum3, mask2, coord2, bfc.reshape(1, HH), W1, b1.reshape(1, HH),
    W2, b2.reshape(1, HH), Wc, bc.reshape(1, D))


def kernel(input_obs_node_gpt2_token, input_obs_node_gpt2_token_mask,
           input_obs_char_obj_rel_gpt2_token, wte, Wfc, bfc,
           W1, b1, W2, b2, Wc, bc):
  tok2 = input_obs_node_gpt2_token.reshape(S, L).astype(jnp.int32)
  mask2 = input_obs_node_gpt2_token_mask.reshape(S, L)
  coord2 = input_obs_char_obj_rel_gpt2_token.reshape(S, 3)

  tok24f = jnp.pad(tok2, ((0, 0), (0, 4))).reshape(-1)
  maskf = jnp.pad(mask2, ((0, 0), (0, 4))).reshape(-1)
  proj3 = _proj_matmul(wte, Wfc)
  ssum3 = _sc_gather_pool(proj3, tok24f, maskf)
  out = _final_stage(ssum3, mask2, coord2, bfc, W1, b1, W2, b2, Wc, bc)
  return out.reshape(B, N, D)


# speedup vs baseline: 1.4783x; 1.2429x over previous
import functools

import jax
import jax.numpy as jnp
from jax import lax
from jax.experimental import pallas as pl
from jax.experimental.pallas import tpu as pltpu
from jax.experimental.pallas import tpu_sc as plsc

B, N, L = 1024, 26, 20
V, D, H = 100000, 768, 512
HH = H // 2
S = B * N

NC, NS, LANES = 2, 16, 16
NW = NC * NS
SPW = S // NW
CHSC = 4


def _dot3(a, b):
  a_hi = a.astype(jnp.bfloat16)
  a_lo = (a - a_hi.astype(jnp.float32)).astype(jnp.bfloat16)
  b_hi = b.astype(jnp.bfloat16)
  b_lo = (b - b_hi.astype(jnp.float32)).astype(jnp.bfloat16)
  f = functools.partial(jnp.dot, preferred_element_type=jnp.float32)
  return f(a_hi, b_hi) + f(a_lo, b_hi) + f(a_hi, b_lo)


def _proj_matmul(wte, Wfc):
  BM = 2000

  def body(wte_ref, wfc_ref, out_ref):
    out_ref[...] = _dot3(wte_ref[...], wfc_ref[...]).reshape(BM, 2, 128)

  return pl.pallas_call(
      body,
      grid=(V // BM,),
      in_specs=[
          pl.BlockSpec((BM, D), lambda i: (i, 0)),
          pl.BlockSpec((D, HH), lambda i: (0, 0)),
      ],
      out_specs=pl.BlockSpec((BM, 2, 128), lambda i: (i, 0, 0)),
      out_shape=jax.ShapeDtypeStruct((V, 2, 128), jnp.float32),
  )(wte, Wfc)


def _lane_bcast(vec, lane):
  idx = jnp.full((LANES, 1), lane, jnp.int32)
  dnums = lax.GatherDimensionNumbers(
      offset_dims=(), collapsed_slice_dims=(0,), start_index_map=(0,))
  return lax.gather(vec, idx, dnums, (1,),
                    mode=lax.GatherScatterMode.PROMISE_IN_BOUNDS)


def _sc_gather_pool(proj3, tok24f, maskf):
  mesh = plsc.VectorSubcoreMesh(core_axis_name="c", subcore_axis_name="s")
  NCH = SPW // CHSC

  @functools.partial(
      pl.kernel,
      out_type=jax.ShapeDtypeStruct((S, 2, 128), jnp.float32),
      mesh=mesh,
      scratch_types=[
          pltpu.VMEM((SPW * 24,), jnp.int32),
          pltpu.VMEM((SPW * 24 + 8,), jnp.float32),
          pltpu.VMEM((CHSC, L, 2, 128), jnp.float32),
          pltpu.VMEM((CHSC, L, 2, 128), jnp.float32),
          pltpu.VMEM((CHSC, 2, 128), jnp.float32),
          pltpu.SemaphoreType.DMA,
          pltpu.SemaphoreType.DMA,
      ],
  )
  def sc_kernel(proj_hbm, tok_hbm, mask_hbm, out_hbm,
                tok_v, mask_v, rows_a, rows_b, ostage, sem_a, sem_b):
    wid = lax.axis_index("s") * NC + lax.axis_index("c")
    base = wid * SPW
    pltpu.sync_copy(tok_hbm.at[pl.ds(base * 24, SPW * 24)], tok_v)
    pltpu.sync_copy(mask_hbm.at[pl.ds(base * 24, SPW * 24)],
                    mask_v.at[pl.ds(0, SPW * 24)])

    def issue(c, buf, sem):
      def body(k, carry):
        idx = tok_v.at[pl.ds((c * CHSC + k) * 24, L)]
        pltpu.async_copy(proj_hbm.at[idx], buf.at[k], sem)
        return carry
      lax.fori_loop(0, CHSC, body, 0)

    def drain(buf, sem):
      def body(k, carry):
        idx = tok_v.at[pl.ds(0, L)]
        pltpu.make_async_copy(proj_hbm.at[idx], buf.at[k], sem).wait()
        return carry
      lax.fori_loop(0, CHSC, body, 0)

    def compute(c, buf):
      def seg_body(si, carry):
        moff = (c * CHSC + si) * 24
        m0 = mask_v[pl.ds(moff, LANES)]
        m1 = mask_v[pl.ds(moff + LANES, LANES)]
        acc = [jnp.zeros((LANES,), jnp.float32) for _ in range(HH // LANES)]
        for l in range(L):
          if l < LANES:
            w = _lane_bcast(m0, l)
          else:
            w = _lane_bcast(m1, l - LANES)
          for j in range(HH // LANES):
            acc[j] = acc[j] + w * buf[si, l, j // 8,
                                      pl.ds((j % 8) * LANES, LANES)]
        for j in range(HH // LANES):
          ostage[si, j // 8, pl.ds((j % 8) * LANES, LANES)] = acc[j]
        return carry

      lax.fori_loop(0, CHSC, seg_body, 0)
      pltpu.sync_copy(ostage, out_hbm.at[pl.ds(base + c * CHSC, CHSC)])

    issue(0, rows_a, sem_a)
    issue(1, rows_b, sem_b)

    def pair_body(p, carry):
      ca = 2 * p
      drain(rows_a, sem_a)
      compute(ca, rows_a)
      issue(ca + 2, rows_a, sem_a)
      drain(rows_b, sem_b)
      compute(ca + 1, rows_b)
      issue(ca + 3, rows_b, sem_b)
      return carry

    lax.fori_loop(0, NCH // 2 - 1, pair_body, 0)
    drain(rows_a, sem_a)
    compute(NCH - 2, rows_a)
    drain(rows_b, sem_b)
    compute(NCH - 1, rows_b)

  return sc_kernel(proj3, tok24f, maskf)


def _final_stage(ssum3, mask2, coord2, bfc, W1, b1, W2, b2, Wc, bc):
  R = 512

  def body(ssum_ref, mask_ref, coord_ref, bfc_ref, w1_ref, b1_ref,
           w2_ref, b2_ref, wc_ref, bc_ref, out_ref):
    d = jnp.sum(mask_ref[...], axis=1, keepdims=True) + 1e-9
    cls = ssum_ref[...].reshape(R, HH) / d + bfc_ref[...]
    c = coord_ref[...]
    w1 = w1_ref[...]
    h = (c[:, 0:1] * w1[0:1, :] + c[:, 1:2] * w1[1:2, :]
         + c[:, 2:3] * w1[2:3, :] + b1_ref[...])
    h = jnp.maximum(h, 0.0)
    ce = _dot3(h, w2_ref[...]) + b2_ref[...]
    wc = wc_ref[...]
    out_ref[...] = (_dot3(jnp.maximum(cls, 0.0), wc[:HH])
                    + _dot3(jnp.maximum(ce, 0.0), wc[HH:])
                    + bc_ref[...])

  return pl.pallas_call(
      body,
      grid=(S // R,),
      in_specs=[
          pl.BlockSpec((R, 2, 128), lambda i: (i, 0, 0)),
          pl.BlockSpec((R, L), lambda i: (i, 0)),
          pl.BlockSpec((R, 3), lambda i: (i, 0)),
          pl.BlockSpec((1, HH), lambda i: (0, 0)),
          pl.BlockSpec((3, HH), lambda i: (0, 0)),
          pl.BlockSpec((1, HH), lambda i: (0, 0)),
          pl.BlockSpec((HH, HH), lambda i: (0, 0)),
          pl.BlockSpec((1, HH), lambda i: (0, 0)),
          pl.BlockSpec((H, D), lambda i: (0, 0)),
          pl.BlockSpec((1, D), lambda i: (0, 0)),
      ],
      out_specs=pl.BlockSpec((R, D), lambda i: (i, 0)),
      out_shape=jax.ShapeDtypeStruct((S, D), jnp.float32),
  )(ssum3, mask2, coord2, bfc.reshape(1, HH), W1, b1.reshape(1, HH),
    W2, b2.reshape(1, HH), Wc, bc.reshape(1, D))


def kernel(input_obs_node_gpt2_token, input_obs_node_gpt2_token_mask,
           input_obs_char_obj_rel_gpt2_token, wte, Wfc, bfc,
           W1, b1, W2, b2, Wc, bc):
  tok2 = input_obs_node_gpt2_token.reshape(S, L).astype(jnp.int32)
  mask2 = input_obs_node_gpt2_token_mask.reshape(S, L)
  coord2 = input_obs_char_obj_rel_gpt2_token.reshape(S, 3)

  tok24f = jnp.pad(tok2, ((0, 0), (0, 4))).reshape(-1)
  maskf = jnp.pad(mask2, ((0, 0), (0, 4))).reshape(-1)
  proj3 = _proj_matmul(wte, Wfc)
  ssum3 = _sc_gather_pool(proj3, tok24f, maskf)
  out = _final_stage(ssum3, mask2, coord2, bfc, W1, b1, W2, b2, Wc, bc)
  return out.reshape(B, N, D)
